# Initial kernel scaffold; baseline (speedup 1.0000x reference)
import jax
import jax.numpy as jnp
from jax.experimental import pallas as pl


def kernel(token_ids, weight):
    raise NotImplementedError("write your pallas kernel here")


# speedup vs baseline: 1.7980x; 1.7980x over previous
import functools

import jax
import jax.numpy as jnp
from jax import lax
from jax.experimental import pallas as pl
from jax.experimental.pallas import tpu as pltpu
from jax.experimental.pallas import tpu_sc as plsc

_NUM_CORES = 2
_NUM_SUBCORES = 16
_NUM_WORKERS = _NUM_CORES * _NUM_SUBCORES

_IDX_W = 128
_IDX_ROWS = 4
_CHUNK = _IDX_ROWS * _IDX_W


def _emb_body(rows_per_w, n_chunks, d, idx_hbm, table_hbm, out_hbm,
              idx_v, rows_v, sem):
    wid = lax.axis_index("s") * _NUM_CORES + lax.axis_index("c")
    row_base = wid * rows_per_w

    def body(i, carry):
        row_off = row_base + i * _IDX_ROWS
        pltpu.sync_copy(idx_hbm.at[pl.ds(row_off, _IDX_ROWS)], idx_v)
        for j in range(_IDX_ROWS):
            pltpu.async_copy(table_hbm.at[idx_v.at[j]],
                             rows_v.at[pl.ds(j * _IDX_W, _IDX_W)], sem)
        for j in range(_IDX_ROWS):
            pltpu.make_async_copy(
                table_hbm.at[idx_v.at[j]],
                rows_v.at[pl.ds(j * _IDX_W, _IDX_W)], sem).wait()
        pltpu.sync_copy(rows_v, out_hbm.at[pl.ds(row_off * _IDX_W, _CHUNK)])
        return carry

    lax.fori_loop(0, n_chunks, body, 0)


def kernel(token_ids, weight):
    b, s = token_ids.shape
    v, d = weight.shape
    n = b * s
    idx = token_ids.reshape(n // _IDX_W, _IDX_W).astype(jnp.int32)

    rows_per_w = (n // _IDX_W) // _NUM_WORKERS
    n_chunks = rows_per_w // _IDX_ROWS

    mesh = plsc.VectorSubcoreMesh(
        core_axis_name="c", subcore_axis_name="s",
        num_cores=_NUM_CORES, num_subcores=_NUM_SUBCORES)

    emb = functools.partial(
        pl.kernel,
        out_type=jax.ShapeDtypeStruct((n, d), jnp.float32),
        mesh=mesh,
        compiler_params=pltpu.CompilerParams(use_tc_tiling_on_sc=False),
        scratch_types=[
            pltpu.VMEM((_IDX_ROWS, _IDX_W), jnp.int32),
            pltpu.VMEM((_CHUNK, d), jnp.float32),
            pltpu.SemaphoreType.DMA,
        ],
    )(functools.partial(_emb_body, rows_per_w, n_chunks, d))

    out = emb(idx, weight)
    return out.reshape(b, s, d)


# speedup vs baseline: 1.8712x; 1.0407x over previous
import functools

import jax
import jax.numpy as jnp
from jax import lax
from jax.experimental import pallas as pl
from jax.experimental.pallas import tpu as pltpu
from jax.experimental.pallas import tpu_sc as plsc

_NUM_CORES = 2
_NUM_SUBCORES = 16
_NUM_WORKERS = _NUM_CORES * _NUM_SUBCORES

_IDX_W = 128
_IDX_ROWS = 4
_CHUNK = _IDX_ROWS * _IDX_W


def _emb_body(rows_per_w, n_chunks, d, idx_hbm, table_hbm, out_hbm,
              idx_v, rows_v, gat_sem, out_sem):
    wid = lax.axis_index("s") * _NUM_CORES + lax.axis_index("c")
    row_base = wid * rows_per_w

    def fire(i):
        buf = lax.rem(i, 2)
        for j in range(_IDX_ROWS):
            pltpu.async_copy(
                table_hbm.at[idx_v.at[i * _IDX_ROWS + j]],
                rows_v.at[buf].at[pl.ds(j * _IDX_W, _IDX_W)], gat_sem)

    def wait_gather(i):
        buf = lax.rem(i, 2)
        for j in range(_IDX_ROWS):
            pltpu.make_async_copy(
                table_hbm.at[idx_v.at[0]],
                rows_v.at[buf].at[pl.ds(j * _IDX_W, _IDX_W)], gat_sem).wait()

    def out_slice(i):
        return out_hbm.at[pl.ds((row_base + i * _IDX_ROWS) * _IDX_W, _CHUNK)]

    def writeout(i):
        pltpu.async_copy(rows_v.at[lax.rem(i, 2)], out_slice(i), out_sem)

    def wait_out(i):
        pltpu.make_async_copy(rows_v.at[0], out_slice(i), out_sem).wait()

    pltpu.sync_copy(idx_hbm.at[pl.ds(row_base, rows_per_w)], idx_v)

    fire(0)
    wait_gather(0)
    writeout(0)
    fire(1)

    def body(i, carry):
        wait_gather(i)
        wait_out(i - 1)
        writeout(i)
        fire(i + 1)
        return carry

    lax.fori_loop(1, n_chunks - 1, body, 0)

    i_last = n_chunks - 1
    wait_gather(i_last)
    wait_out(i_last - 1)
    writeout(i_last)
    wait_out(i_last)


def kernel(token_ids, weight):
    b, s = token_ids.shape
    v, d = weight.shape
    n = b * s
    idx = token_ids.reshape(n // _IDX_W, _IDX_W).astype(jnp.int32)

    rows_per_w = (n // _IDX_W) // _NUM_WORKERS
    n_chunks = rows_per_w // _IDX_ROWS

    mesh = plsc.VectorSubcoreMesh(
        core_axis_name="c", subcore_axis_name="s",
        num_cores=_NUM_CORES, num_subcores=_NUM_SUBCORES)

    emb = functools.partial(
        pl.kernel,
        out_type=jax.ShapeDtypeStruct((n, d), jnp.float32),
        mesh=mesh,
        compiler_params=pltpu.CompilerParams(use_tc_tiling_on_sc=False),
        scratch_types=[
            pltpu.VMEM((rows_per_w, _IDX_W), jnp.int32),
            pltpu.VMEM((2, _CHUNK, d), jnp.float32),
            pltpu.SemaphoreType.DMA,
            pltpu.SemaphoreType.DMA,
        ],
    )(functools.partial(_emb_body, rows_per_w, n_chunks, d))

    out = emb(idx, weight)
    return out.reshape(b, s, d)
